# Initial kernel scaffold; baseline (speedup 1.0000x reference)
import jax
import jax.numpy as jnp
from jax.experimental import pallas as pl


def kernel(x, mask):
    raise NotImplementedError("write your pallas kernel here")


# speedup vs baseline: 4.2649x; 4.2649x over previous
import jax
import jax.numpy as jnp
from jax.experimental import pallas as pl
from jax.experimental.pallas import tpu as pltpu

_BR = 1024
_BC = 512
_N_ROWS = 16384
_N_COLS = 4096
_K = 128


def _tc_body(mask_ref, x_ref, out_ref):
    m = mask_ref[0, 0, :]
    iota = jax.lax.broadcasted_iota(jnp.int32, (_K, _BC), 0)
    onehot = (iota == m[None, :]).astype(jnp.float32)
    out_ref[...] = jnp.dot(x_ref[...], onehot,
                           preferred_element_type=jnp.float32)


def kernel(x, mask):
    n_row_tiles = _N_ROWS // _BR
    n_col_tiles = _N_COLS // _BC
    mask3 = mask.reshape(n_col_tiles, 1, _BC)
    return pl.pallas_call(
        _tc_body,
        grid=(n_row_tiles, n_col_tiles),
        in_specs=[
            pl.BlockSpec((1, 1, _BC), lambda i, j: (j, 0, 0)),
            pl.BlockSpec((_BR, _K), lambda i, j: (i, 0)),
        ],
        out_specs=pl.BlockSpec((_BR, _BC), lambda i, j: (i, j)),
        out_shape=jax.ShapeDtypeStruct((_N_ROWS, _N_COLS), jnp.float32),
    )(mask3, x)
